# Initial kernel scaffold; baseline (speedup 1.0000x reference)
import jax
import jax.numpy as jnp
from jax.experimental import pallas as pl


def kernel(x, edge_index, edge_attr, edge_weight, W1, b1, Wk1, bk1, Wk2, bk2, W2, b2, g_in, beta_in, g_msg, beta_msg, g_upd, beta_upd, g_out, beta_out):
    raise NotImplementedError("write your pallas kernel here")


# speedup vs baseline: 1.3994x; 1.3994x over previous
import functools

import jax
import jax.numpy as jnp
from jax import lax
from jax.experimental import pallas as pl
from jax.experimental.pallas import tpu as pltpu
from jax.experimental.pallas import tpu_sc as plsc

_EPS = 1e-5
_CHUNK = 128


def _pre_body(x_ref, g_ref, b_ref, w1t_ref, b1_ref, y_ref):
    X = x_ref[...]
    m = jnp.mean(X, axis=0, keepdims=True)
    dv = X - m
    v = jnp.mean(dv * dv, axis=0, keepdims=True)
    h = dv * (g_ref[...] * lax.rsqrt(v + _EPS)) + b_ref[...]
    h = jnp.maximum(h, 0.0)
    y_ref[...] = jnp.dot(h, w1t_ref[...], preferred_element_type=jnp.float32) + b1_ref[...]


def _stats_body(msg_ref, out_ref, *, block_e, n_edges):
    i = pl.program_id(0)

    @pl.when(i == 0)
    def _():
        out_ref[...] = jnp.zeros_like(out_ref)

    rows = i * block_e + lax.broadcasted_iota(jnp.int32, (block_e, 1), 0)
    Mv = jnp.where(rows < n_edges, msg_ref[...], 0.0)
    s1 = jnp.sum(Mv, axis=0, keepdims=True)
    s2 = jnp.sum(Mv * Mv, axis=0, keepdims=True)
    out_ref[...] += jnp.concatenate([s1, s2], axis=0)


def _edge_body(ea_ref, msg_ref, ew_ref, st_ref, gm_ref, bm_ref, wk1t_ref,
               bk1_ref, w0_ref, b0_ref, tsel_ref, bsel_ref, out_ref, *,
               block_e, n_edges, d_hid):
    i = pl.program_id(0)
    m = st_ref[0:1, :] * (1.0 / n_edges)
    v = st_ref[1:2, :] * (1.0 / n_edges) - m * m
    a = gm_ref[...] * lax.rsqrt(v + _EPS)
    c = bm_ref[...] - m * a
    msgn = jnp.maximum(msg_ref[...] * a + c, 0.0)
    t = jnp.dot(ea_ref[...], wk1t_ref[...], preferred_element_type=jnp.float32)
    t = jnp.maximum(t + bk1_ref[...], 0.0)
    acc = jnp.dot(t, w0_ref[...], preferred_element_type=jnp.float32) + b0_ref[...]
    for j in range(d_hid):
        kj = jnp.dot(t, tsel_ref[j], preferred_element_type=jnp.float32) + bsel_ref[j:j + 1, :]
        acc = acc + kj * msgn[:, j:j + 1]
    acc = acc * ew_ref[...]
    rows = i * block_e + lax.broadcasted_iota(jnp.int32, (block_e, 1), 0)
    out_ref[...] = jnp.where(rows < n_edges, acc, 0.0)


def _post_body(u_ref, gu_ref, bu_ref, w2t_ref, b2_ref, go_ref, bo_ref, out_ref):
    U = u_ref[0] + u_ref[1]
    m = jnp.mean(U, axis=0, keepdims=True)
    dv = U - m
    v = jnp.mean(dv * dv, axis=0, keepdims=True)
    un = jnp.maximum(dv * (gu_ref[...] * lax.rsqrt(v + _EPS)) + bu_ref[...], 0.0)
    O = jnp.dot(un, w2t_ref[...], preferred_element_type=jnp.float32) + b2_ref[...]
    m2 = jnp.mean(O, axis=0, keepdims=True)
    d2 = O - m2
    v2 = jnp.mean(d2 * d2, axis=0, keepdims=True)
    out_ref[...] = d2 * (go_ref[...] * lax.rsqrt(v2 + _EPS)) + bo_ref[...]


@functools.lru_cache(maxsize=None)
def _make_gather(n_nodes, d, ep):
    info = plsc.get_sparse_core_info()
    nc, ns = info.num_cores, info.num_subcores
    nw = nc * ns
    pw = ep // nw
    ch = pw // _CHUNK
    mesh = plsc.VectorSubcoreMesh(core_axis_name="c", subcore_axis_name="s")

    @functools.partial(
        pl.kernel,
        mesh=mesh,
        out_type=jax.ShapeDtypeStruct((ep, d), jnp.float32),
        scratch_types=[
            pltpu.VMEM((ch, _CHUNK), jnp.int32),
            pltpu.VMEM((pw, d), jnp.float32),
            pltpu.SemaphoreType.DMA,
        ],
        compiler_params=pltpu.CompilerParams(use_tc_tiling_on_sc=False),
    )
    def gather_k(y_hbm, idx_hbm, out_hbm, idx_v, rows_v, sem):
        cc = lax.axis_index("c")
        ss = lax.axis_index("s")
        wid = ss * nc + cc
        pltpu.sync_copy(idx_hbm.at[pl.ds(wid * ch, ch)], idx_v)

        def body(j, carry):
            pltpu.async_copy(y_hbm.at[idx_v.at[j]],
                             rows_v.at[pl.ds(j * _CHUNK, _CHUNK)], sem)
            return carry

        lax.fori_loop(0, ch, body, 0)
        pltpu.make_async_copy(out_hbm.at[pl.ds(wid * pw, pw)], rows_v, sem).wait()
        pltpu.sync_copy(rows_v, out_hbm.at[pl.ds(wid * pw, pw)])

    return gather_k


@functools.lru_cache(maxsize=None)
def _make_scatter(n_nodes, d, ep):
    info = plsc.get_sparse_core_info()
    nc, ns = info.num_cores, info.num_subcores
    nw = nc * ns
    pw = ep // nw
    ch = pw // _CHUNK
    rps = n_nodes // ns
    mesh = plsc.VectorSubcoreMesh(core_axis_name="c", subcore_axis_name="s")

    @functools.partial(
        pl.kernel,
        mesh=mesh,
        out_type=jax.ShapeDtypeStruct((nc, n_nodes, d), jnp.float32),
        scratch_types=[
            pltpu.VMEM((ch, _CHUNK), jnp.int32),
            pltpu.VMEM((pw, d), jnp.float32),
            pltpu.VMEM_SHARED((n_nodes, d), jnp.float32),
            pltpu.SemaphoreType.DMA,
        ],
        compiler_params=pltpu.CompilerParams(use_tc_tiling_on_sc=False),
    )
    def scatter_k(contrib_hbm, idx_hbm, zeros_hbm, out_hbm, idx_v, cont_v, acc_sh, sem):
        cc = lax.axis_index("c")
        ss = lax.axis_index("s")
        wid = ss * nc + cc

        @pl.when(ss == 0)
        def _():
            pltpu.sync_copy(zeros_hbm, acc_sh)

        pltpu.sync_copy(idx_hbm.at[pl.ds(wid * ch, ch)], idx_v)
        pltpu.sync_copy(contrib_hbm.at[pl.ds(wid * pw, pw)], cont_v)
        plsc.subcore_barrier()

        def body(j, carry):
            pltpu.sync_copy(cont_v.at[pl.ds(j * _CHUNK, _CHUNK)],
                            acc_sh.at[idx_v.at[j]], add=True)
            return carry

        lax.fori_loop(0, ch, body, 0)
        plsc.subcore_barrier()
        pltpu.sync_copy(acc_sh.at[pl.ds(ss * rps, rps)],
                        out_hbm.at[cc].at[pl.ds(ss * rps, rps)])

    return scatter_k


def kernel(x, edge_index, edge_attr, edge_weight, W1, b1, Wk1, bk1, Wk2, bk2,
           W2, b2, g_in, beta_in, g_msg, beta_msg, g_upd, beta_upd, g_out, beta_out):
    n, d_in = x.shape
    e, d_edge = edge_attr.shape
    d_hid = W1.shape[0]
    d_out = W2.shape[0]
    k_hid = Wk1.shape[0]

    info = plsc.get_sparse_core_info()
    nw = info.num_cores * info.num_subcores
    unit = nw * _CHUNK
    ep = ((e + unit - 1) // unit) * unit

    idx_in = jnp.pad(edge_index[0].astype(jnp.int32), (0, ep - e)).reshape(ep // _CHUNK, _CHUNK)
    idx_out = jnp.pad(edge_index[1].astype(jnp.int32), (0, ep - e)).reshape(ep // _CHUNK, _CHUNK)

    y = pl.pallas_call(
        _pre_body,
        out_shape=jax.ShapeDtypeStruct((n, d_hid), jnp.float32),
        in_specs=[
            pl.BlockSpec((n, d_in), lambda: (0, 0)),
            pl.BlockSpec((1, d_in), lambda: (0, 0)),
            pl.BlockSpec((1, d_in), lambda: (0, 0)),
            pl.BlockSpec((d_in, d_hid), lambda: (0, 0)),
            pl.BlockSpec((1, d_hid), lambda: (0, 0)),
        ],
        out_specs=pl.BlockSpec((n, d_hid), lambda: (0, 0)),
    )(x, g_in.reshape(1, -1), beta_in.reshape(1, -1), W1.T, b1.reshape(1, -1))

    msg_raw = _make_gather(n, d_hid, ep)(y, idx_in)

    block_e = 4096
    grid = ep // block_e
    stats = pl.pallas_call(
        functools.partial(_stats_body, block_e=block_e, n_edges=e),
        grid=(grid,),
        out_shape=jax.ShapeDtypeStruct((2, d_hid), jnp.float32),
        in_specs=[pl.BlockSpec((block_e, d_hid), lambda i: (i, 0))],
        out_specs=pl.BlockSpec((2, d_hid), lambda i: (0, 0)),
    )(msg_raw)

    w0 = Wk2[:d_hid].T
    b0 = bk2[:d_hid].reshape(1, -1)
    tsel = Wk2.reshape(d_hid + 1, d_hid, k_hid)[1:].transpose(1, 2, 0)
    bsel = bk2.reshape(d_hid + 1, d_hid)[1:].T
    contrib = pl.pallas_call(
        functools.partial(_edge_body, block_e=block_e, n_edges=e, d_hid=d_hid),
        grid=(grid,),
        out_shape=jax.ShapeDtypeStruct((ep, d_hid), jnp.float32),
        in_specs=[
            pl.BlockSpec((block_e, d_edge), lambda i: (i, 0)),
            pl.BlockSpec((block_e, d_hid), lambda i: (i, 0)),
            pl.BlockSpec((block_e, 1), lambda i: (i, 0)),
            pl.BlockSpec((2, d_hid), lambda i: (0, 0)),
            pl.BlockSpec((1, d_hid), lambda i: (0, 0)),
            pl.BlockSpec((1, d_hid), lambda i: (0, 0)),
            pl.BlockSpec((d_edge, k_hid), lambda i: (0, 0)),
            pl.BlockSpec((1, k_hid), lambda i: (0, 0)),
            pl.BlockSpec((k_hid, d_hid), lambda i: (0, 0)),
            pl.BlockSpec((1, d_hid), lambda i: (0, 0)),
            pl.BlockSpec((d_hid, k_hid, d_hid), lambda i: (0, 0, 0)),
            pl.BlockSpec((d_hid, d_hid), lambda i: (0, 0)),
        ],
        out_specs=pl.BlockSpec((block_e, d_hid), lambda i: (i, 0)),
    )(edge_attr, msg_raw, edge_weight.reshape(-1, 1), stats,
      g_msg.reshape(1, -1), beta_msg.reshape(1, -1), Wk1.T,
      bk1.reshape(1, -1), w0, b0, tsel, bsel)

    zeros = jnp.zeros((n, d_hid), jnp.float32)
    parts = _make_scatter(n, d_hid, ep)(contrib, idx_out, zeros)

    out = pl.pallas_call(
        _post_body,
        out_shape=jax.ShapeDtypeStruct((n, d_out), jnp.float32),
        in_specs=[
            pl.BlockSpec((2, n, d_hid), lambda: (0, 0, 0)),
            pl.BlockSpec((1, d_hid), lambda: (0, 0)),
            pl.BlockSpec((1, d_hid), lambda: (0, 0)),
            pl.BlockSpec((d_hid, d_out), lambda: (0, 0)),
            pl.BlockSpec((1, d_out), lambda: (0, 0)),
            pl.BlockSpec((1, d_out), lambda: (0, 0)),
            pl.BlockSpec((1, d_out), lambda: (0, 0)),
        ],
        out_specs=pl.BlockSpec((n, d_out), lambda: (0, 0)),
    )(parts, g_upd.reshape(1, -1), beta_upd.reshape(1, -1), W2.T,
      b2.reshape(1, -1), g_out.reshape(1, -1), beta_out.reshape(1, -1))
    return out


# speedup vs baseline: 3.1880x; 2.2781x over previous
import functools

import jax
import jax.numpy as jnp
from jax import lax
from jax.experimental import pallas as pl
from jax.experimental.pallas import tpu as pltpu
from jax.experimental.pallas import tpu_sc as plsc

_EPS = 1e-5
_CHUNK = 128


def _pre_body(x_ref, g_ref, b_ref, w1t_ref, b1_ref, y_ref):
    X = x_ref[...]
    m = jnp.mean(X, axis=0, keepdims=True)
    dv = X - m
    v = jnp.mean(dv * dv, axis=0, keepdims=True)
    h = dv * (g_ref[...] * lax.rsqrt(v + _EPS)) + b_ref[...]
    h = jnp.maximum(h, 0.0)
    y_ref[...] = jnp.dot(h, w1t_ref[...], preferred_element_type=jnp.float32) + b1_ref[...]


def _stats_body(msg_ref, out_ref, *, block_e, n_edges):
    i = pl.program_id(0)

    @pl.when(i == 0)
    def _():
        out_ref[...] = jnp.zeros_like(out_ref)

    cols = i * block_e + lax.broadcasted_iota(jnp.int32, (16, block_e), 1)
    Mv = jnp.where(cols < n_edges, msg_ref[...], 0.0)
    s1 = jnp.sum(Mv, axis=1, keepdims=True)
    s2 = jnp.sum(Mv * Mv, axis=1, keepdims=True)
    out_ref[...] += jnp.concatenate([s1, s2], axis=1)


def _edge_body(ea_ref, msg_ref, ew_ref, st_ref, gm_ref, bm_ref, wk1_ref,
               bk1_ref, wk2p_ref, bk2p_ref, out_ref, *, block_e, n_edges, d_hid):
    i = pl.program_id(0)
    m = st_ref[:, 0:1] * (1.0 / n_edges)
    v = st_ref[:, 1:2] * (1.0 / n_edges) - m * m
    a = gm_ref[...] * lax.rsqrt(v + _EPS)
    c = bm_ref[...] - m * a
    msgn = jnp.maximum(msg_ref[...] * a + c, 0.0)
    t = jnp.dot(wk1_ref[...], ea_ref[...], preferred_element_type=jnp.float32)
    t = jnp.maximum(t + bk1_ref[...], 0.0)
    kt = jnp.dot(wk2p_ref[...], t, preferred_element_type=jnp.float32) + bk2p_ref[...]
    acc = kt[0:d_hid, :]
    for j in range(d_hid):
        acc = acc + kt[(j + 1) * d_hid:(j + 2) * d_hid, :] * msgn[j:j + 1, :]
    acc = acc * ew_ref[...]
    cols = i * block_e + lax.broadcasted_iota(jnp.int32, (d_hid, block_e), 1)
    out_ref[...] = jnp.where(cols < n_edges, acc, 0.0)


def _post_body(u_ref, gu_ref, bu_ref, w2t_ref, b2_ref, go_ref, bo_ref, out_ref):
    U = u_ref[0] + u_ref[1]
    m = jnp.mean(U, axis=0, keepdims=True)
    dv = U - m
    v = jnp.mean(dv * dv, axis=0, keepdims=True)
    un = jnp.maximum(dv * (gu_ref[...] * lax.rsqrt(v + _EPS)) + bu_ref[...], 0.0)
    O = jnp.dot(un, w2t_ref[...], preferred_element_type=jnp.float32) + b2_ref[...]
    m2 = jnp.mean(O, axis=0, keepdims=True)
    d2 = O - m2
    v2 = jnp.mean(d2 * d2, axis=0, keepdims=True)
    out_ref[...] = d2 * (go_ref[...] * lax.rsqrt(v2 + _EPS)) + bo_ref[...]


@functools.lru_cache(maxsize=None)
def _make_gather(n_nodes, d, ep):
    info = plsc.get_sparse_core_info()
    nc, ns = info.num_cores, info.num_subcores
    nw = nc * ns
    pw = ep // nw
    ch = pw // _CHUNK
    mesh = plsc.VectorSubcoreMesh(core_axis_name="c", subcore_axis_name="s")

    @functools.partial(
        pl.kernel,
        mesh=mesh,
        out_type=jax.ShapeDtypeStruct((ep, d), jnp.float32),
        scratch_types=[
            pltpu.VMEM((ch, _CHUNK), jnp.int32),
            pltpu.VMEM((pw, d), jnp.float32),
            pltpu.SemaphoreType.DMA,
        ],
        compiler_params=pltpu.CompilerParams(use_tc_tiling_on_sc=False),
    )
    def gather_k(y_hbm, idx_hbm, out_hbm, idx_v, rows_v, sem):
        cc = lax.axis_index("c")
        ss = lax.axis_index("s")
        wid = ss * nc + cc
        pltpu.sync_copy(idx_hbm.at[pl.ds(wid * ch, ch)], idx_v)

        def body(j, carry):
            pltpu.async_copy(y_hbm.at[idx_v.at[j]],
                             rows_v.at[pl.ds(j * _CHUNK, _CHUNK)], sem)
            return carry

        lax.fori_loop(0, ch, body, 0)
        pltpu.make_async_copy(out_hbm.at[pl.ds(wid * pw, pw)], rows_v, sem).wait()
        pltpu.sync_copy(rows_v, out_hbm.at[pl.ds(wid * pw, pw)])

    return gather_k


@functools.lru_cache(maxsize=None)
def _make_scatter(n_nodes, d, ep):
    info = plsc.get_sparse_core_info()
    nc, ns = info.num_cores, info.num_subcores
    nw = nc * ns
    pw = ep // nw
    ch = pw // _CHUNK
    rps = n_nodes // ns
    mesh = plsc.VectorSubcoreMesh(core_axis_name="c", subcore_axis_name="s")

    @functools.partial(
        pl.kernel,
        mesh=mesh,
        out_type=jax.ShapeDtypeStruct((nc, n_nodes, d), jnp.float32),
        scratch_types=[
            pltpu.VMEM((ch, _CHUNK), jnp.int32),
            pltpu.VMEM((pw, d), jnp.float32),
            pltpu.VMEM_SHARED((n_nodes, d), jnp.float32),
            pltpu.SemaphoreType.DMA,
        ],
        compiler_params=pltpu.CompilerParams(use_tc_tiling_on_sc=False),
    )
    def scatter_k(contrib_hbm, idx_hbm, zeros_hbm, out_hbm, idx_v, cont_v, acc_sh, sem):
        cc = lax.axis_index("c")
        ss = lax.axis_index("s")
        wid = ss * nc + cc

        @pl.when(ss == 0)
        def _():
            pltpu.sync_copy(zeros_hbm, acc_sh)

        pltpu.sync_copy(idx_hbm.at[pl.ds(wid * ch, ch)], idx_v)
        pltpu.sync_copy(contrib_hbm.at[pl.ds(wid * pw, pw)], cont_v)
        plsc.subcore_barrier()

        def body(j, carry):
            pltpu.sync_copy(cont_v.at[pl.ds(j * _CHUNK, _CHUNK)],
                            acc_sh.at[idx_v.at[j]], add=True)
            return carry

        lax.fori_loop(0, ch, body, 0)
        plsc.subcore_barrier()
        pltpu.sync_copy(acc_sh.at[pl.ds(ss * rps, rps)],
                        out_hbm.at[cc].at[pl.ds(ss * rps, rps)])

    return scatter_k


def kernel(x, edge_index, edge_attr, edge_weight, W1, b1, Wk1, bk1, Wk2, bk2,
           W2, b2, g_in, beta_in, g_msg, beta_msg, g_upd, beta_upd, g_out, beta_out):
    n, d_in = x.shape
    e, d_edge = edge_attr.shape
    d_hid = W1.shape[0]
    d_out = W2.shape[0]
    k_hid = Wk1.shape[0]

    info = plsc.get_sparse_core_info()
    nw = info.num_cores * info.num_subcores
    unit = nw * _CHUNK
    ep = ((e + unit - 1) // unit) * unit

    idx_in = jnp.pad(edge_index[0].astype(jnp.int32), (0, ep - e)).reshape(ep // _CHUNK, _CHUNK)
    idx_out = jnp.pad(edge_index[1].astype(jnp.int32), (0, ep - e)).reshape(ep // _CHUNK, _CHUNK)

    y = pl.pallas_call(
        _pre_body,
        out_shape=jax.ShapeDtypeStruct((n, d_hid), jnp.float32),
        in_specs=[
            pl.BlockSpec((n, d_in), lambda: (0, 0)),
            pl.BlockSpec((1, d_in), lambda: (0, 0)),
            pl.BlockSpec((1, d_in), lambda: (0, 0)),
            pl.BlockSpec((d_in, d_hid), lambda: (0, 0)),
            pl.BlockSpec((1, d_hid), lambda: (0, 0)),
        ],
        out_specs=pl.BlockSpec((n, d_hid), lambda: (0, 0)),
    )(x, g_in.reshape(1, -1), beta_in.reshape(1, -1), W1.T, b1.reshape(1, -1))

    msg_raw = _make_gather(n, d_hid, ep)(y, idx_in)

    msg_t = msg_raw.T
    ea_t = edge_attr.T
    ew_t = edge_weight.reshape(1, -1)

    block_e = 4096
    grid = ep // block_e
    stats = pl.pallas_call(
        functools.partial(_stats_body, block_e=block_e, n_edges=e),
        grid=(grid,),
        out_shape=jax.ShapeDtypeStruct((d_hid, 2), jnp.float32),
        in_specs=[pl.BlockSpec((d_hid, block_e), lambda i: (0, i))],
        out_specs=pl.BlockSpec((d_hid, 2), lambda i: (0, 0)),
    )(msg_t)

    wk2p = jnp.concatenate(
        [Wk2[:d_hid],
         Wk2[d_hid:].reshape(d_hid, d_hid, k_hid).transpose(1, 0, 2).reshape(d_hid * d_hid, k_hid)],
        axis=0)
    bk2p = jnp.concatenate(
        [bk2[:d_hid], bk2[d_hid:].reshape(d_hid, d_hid).T.reshape(-1)], axis=0)
    contrib_t = pl.pallas_call(
        functools.partial(_edge_body, block_e=block_e, n_edges=e, d_hid=d_hid),
        grid=(grid,),
        out_shape=jax.ShapeDtypeStruct((d_hid, ep), jnp.float32),
        in_specs=[
            pl.BlockSpec((d_edge, block_e), lambda i: (0, i)),
            pl.BlockSpec((d_hid, block_e), lambda i: (0, i)),
            pl.BlockSpec((1, block_e), lambda i: (0, i)),
            pl.BlockSpec((d_hid, 2), lambda i: (0, 0)),
            pl.BlockSpec((d_hid, 1), lambda i: (0, 0)),
            pl.BlockSpec((d_hid, 1), lambda i: (0, 0)),
            pl.BlockSpec((k_hid, d_edge), lambda i: (0, 0)),
            pl.BlockSpec((k_hid, 1), lambda i: (0, 0)),
            pl.BlockSpec(((d_hid + 1) * d_hid, k_hid), lambda i: (0, 0)),
            pl.BlockSpec(((d_hid + 1) * d_hid, 1), lambda i: (0, 0)),
        ],
        out_specs=pl.BlockSpec((d_hid, block_e), lambda i: (0, i)),
    )(ea_t, msg_t, ew_t, stats, g_msg.reshape(-1, 1), beta_msg.reshape(-1, 1),
      Wk1, bk1.reshape(-1, 1), wk2p, bk2p.reshape(-1, 1))

    zeros = jnp.zeros((n, d_hid), jnp.float32)
    parts = _make_scatter(n, d_hid, ep)(contrib_t.T, idx_out, zeros)

    out = pl.pallas_call(
        _post_body,
        out_shape=jax.ShapeDtypeStruct((n, d_out), jnp.float32),
        in_specs=[
            pl.BlockSpec((2, n, d_hid), lambda: (0, 0, 0)),
            pl.BlockSpec((1, d_hid), lambda: (0, 0)),
            pl.BlockSpec((1, d_hid), lambda: (0, 0)),
            pl.BlockSpec((d_hid, d_out), lambda: (0, 0)),
            pl.BlockSpec((1, d_out), lambda: (0, 0)),
            pl.BlockSpec((1, d_out), lambda: (0, 0)),
            pl.BlockSpec((1, d_out), lambda: (0, 0)),
        ],
        out_specs=pl.BlockSpec((n, d_out), lambda: (0, 0)),
    )(parts, g_upd.reshape(1, -1), beta_upd.reshape(1, -1), W2.T,
      b2.reshape(1, -1), g_out.reshape(1, -1), beta_out.reshape(1, -1))
    return out
